# Initial kernel scaffold; baseline (speedup 1.0000x reference)
import jax
import jax.numpy as jnp
from jax.experimental import pallas as pl


def kernel(hidden_states, Wg, W1, W3, W2):
    raise NotImplementedError("write your pallas kernel here")


# speedup vs baseline: 1.1633x; 1.1633x over previous
import functools

import jax
import jax.numpy as jnp
from jax import lax
from jax.experimental import pallas as pl
from jax.experimental.pallas import tpu as pltpu
from jax.experimental.pallas import tpu_sc as plsc

D_MODEL = 1024
FFN = 3584
N_EXP = 8
TOKENS = 2048
PAIRS = 2 * TOKENS
BLK = 128
NBLK = 40
G = NBLK * BLK

NW = 32
TOK_W = TOKENS // NW
CHUNK = 32

@functools.cache
def _sc_mesh():
    return plsc.VectorSubcoreMesh(core_axis_name="c", subcore_axis_name="s")


def _router_body(x_ref, wg_ref, e1_ref, e2_ref, w1_ref, w2_ref):
    x = x_ref[...]
    wg = wg_ref[...]
    logits = lax.dot_general(x, wg, (((1,), (1,)), ((), ())),
                             preferred_element_type=jnp.float32)
    ids = lax.broadcasted_iota(jnp.int32, logits.shape, 1)
    m1 = jnp.max(logits, axis=1, keepdims=True)
    a1 = jnp.min(jnp.where(logits == m1, ids, N_EXP), axis=1, keepdims=True)
    masked = jnp.where(ids == a1, -jnp.inf, logits)
    m2 = jnp.max(masked, axis=1, keepdims=True)
    a2 = jnp.min(jnp.where(masked == m2, ids, N_EXP), axis=1, keepdims=True)
    e1_ref[...] = a1
    e2_ref[...] = a2
    w1_ref[...] = jax.nn.sigmoid(m1 - m2)
    w2_ref[...] = jax.nn.sigmoid(m2 - m1)


def _router(x, Wg):
    rb = 256
    return pl.pallas_call(
        _router_body,
        grid=(TOKENS // rb,),
        in_specs=[pl.BlockSpec((rb, D_MODEL), lambda b: (b, 0)),
                  pl.BlockSpec((N_EXP, D_MODEL), lambda b: (0, 0))],
        out_specs=[pl.BlockSpec((rb, 1), lambda b: (b, 0)),
                   pl.BlockSpec((rb, 1), lambda b: (b, 0)),
                   pl.BlockSpec((rb, 1), lambda b: (b, 0)),
                   pl.BlockSpec((rb, 1), lambda b: (b, 0))],
        out_shape=[jax.ShapeDtypeStruct((TOKENS, 1), jnp.int32),
                   jax.ShapeDtypeStruct((TOKENS, 1), jnp.int32),
                   jax.ShapeDtypeStruct((TOKENS, 1), jnp.float32),
                   jax.ShapeDtypeStruct((TOKENS, 1), jnp.float32)],
    )(x, Wg)


def _metadata(e1, e2):
    ef = jnp.stack([e1, e2], axis=1).reshape(-1)
    onehot = ef[:, None] == jnp.arange(N_EXP)[None, :]
    counts = jnp.sum(onehot.astype(jnp.int32), axis=0)
    nb = (counts + BLK - 1) // BLK
    cumnb = jnp.cumsum(nb)
    astart = jnp.concatenate([jnp.zeros(1, cumnb.dtype), cumnb[:-1]]) * BLK
    ustart = jnp.concatenate([jnp.zeros(1, counts.dtype), jnp.cumsum(counts)[:-1]])
    order = jnp.argsort(ef, stable=True)
    ranks_sorted = jnp.arange(PAIRS, dtype=jnp.int32) - ustart[ef[order]].astype(jnp.int32)
    rank = jnp.zeros(PAIRS, jnp.int32).at[order].set(ranks_sorted)
    pos = astart[ef].astype(jnp.int32) + rank
    pos0 = pos[0::2]
    pos1 = pos[1::2]
    block_expert = jnp.minimum(
        jnp.searchsorted(cumnb, jnp.arange(NBLK), side="right"),
        N_EXP - 1).astype(jnp.int32)
    return pos0, pos1, block_expert


@functools.cache
def _dispatch_kernel():
    @functools.partial(
        pl.kernel, mesh=_sc_mesh(),
        out_type=jax.ShapeDtypeStruct((G, D_MODEL), jnp.float32),
        scratch_types=[pltpu.VMEM((TOK_W,), jnp.int32),
                       pltpu.VMEM((TOK_W,), jnp.int32),
                       pltpu.VMEM((TOK_W, D_MODEL), jnp.float32),
                       pltpu.SemaphoreType.DMA,
                       pltpu.SemaphoreType.DMA])
    def _dispatch(x_hbm, pos0_hbm, pos1_hbm, xs_hbm, idx0_v, idx1_v, rows_v,
                  sem0, sem1):
        wid = lax.axis_index("s") * 2 + lax.axis_index("c")
        t0 = wid * TOK_W
        pltpu.sync_copy(x_hbm.at[pl.ds(t0, TOK_W)], rows_v)
        pltpu.sync_copy(pos0_hbm.at[pl.ds(t0, TOK_W)], idx0_v)
        pltpu.sync_copy(pos1_hbm.at[pl.ds(t0, TOK_W)], idx1_v)
        cp0 = pltpu.async_copy(rows_v, xs_hbm.at[idx0_v], sem0)
        cp1 = pltpu.async_copy(rows_v, xs_hbm.at[idx1_v], sem1)
        cp0.wait()
        cp1.wait()

    return _dispatch


NF = 2
FT = FFN // NF


def _ffn_body(be_ref, x_ref, w1_ref, w3_ref, w2_ref, o_ref):
    x = x_ref[...]
    a = lax.dot_general(x, w1_ref[0], (((1,), (1,)), ((), ())),
                        preferred_element_type=jnp.float32)
    b = lax.dot_general(x, w3_ref[0], (((1,), (1,)), ((), ())),
                        preferred_element_type=jnp.float32)
    h = a * jax.nn.sigmoid(a) * b
    o_ref[0] = lax.dot_general(h, w2_ref[0], (((1,), (1,)), ((), ())),
                               preferred_element_type=jnp.float32)


def _ffn(block_expert, xs, W1, W3, W2):
    grid_spec = pltpu.PrefetchScalarGridSpec(
        num_scalar_prefetch=1,
        grid=(NF, NBLK),
        in_specs=[
            pl.BlockSpec((BLK, D_MODEL), lambda f, b, be: (b, 0)),
            pl.BlockSpec((1, FT, D_MODEL), lambda f, b, be: (be[b], f, 0)),
            pl.BlockSpec((1, FT, D_MODEL), lambda f, b, be: (be[b], f, 0)),
            pl.BlockSpec((1, D_MODEL, FT), lambda f, b, be: (be[b], 0, f)),
        ],
        out_specs=pl.BlockSpec((1, BLK, D_MODEL), lambda f, b, be: (f, b, 0)),
    )
    ys = pl.pallas_call(
        _ffn_body,
        grid_spec=grid_spec,
        out_shape=jax.ShapeDtypeStruct((NF, G, D_MODEL), jnp.float32),
    )(block_expert, xs, W1, W3, W2)
    return ys.reshape(NF * G, D_MODEL)


@functools.cache
def _combine_kernel():
    @functools.partial(
        pl.kernel, mesh=_sc_mesh(),
        out_type=jax.ShapeDtypeStruct((TOKENS, D_MODEL), jnp.float32),
        scratch_types=[pltpu.VMEM((CHUNK,), jnp.int32),
                       pltpu.VMEM((CHUNK,), jnp.int32),
                       pltpu.VMEM((CHUNK,), jnp.int32),
                       pltpu.VMEM((CHUNK,), jnp.float32),
                       pltpu.VMEM((CHUNK,), jnp.float32),
                       pltpu.VMEM((CHUNK, D_MODEL), jnp.float32),
                       pltpu.VMEM((CHUNK, D_MODEL), jnp.float32),
                       pltpu.VMEM((CHUNK, D_MODEL), jnp.float32),
                       pltpu.SemaphoreType.DMA,
                       pltpu.SemaphoreType.DMA])
    def _combine(ys_hbm, pos0_hbm, pos1_hbm, w0_hbm, w1_hbm, out_hbm,
                 p0_v, p1_v, idx_v, w0_v, w1_v, acc_v, tmpa_v, tmpb_v,
                 sema, semb):
        wid = lax.axis_index("s") * 2 + lax.axis_index("c")
        for chunk in range(TOK_W // CHUNK):
            t0 = wid * TOK_W + chunk * CHUNK
            pltpu.sync_copy(pos0_hbm.at[pl.ds(t0, CHUNK)], p0_v)
            pltpu.sync_copy(pos1_hbm.at[pl.ds(t0, CHUNK)], p1_v)
            pltpu.sync_copy(w0_hbm.at[pl.ds(t0, CHUNK)], w0_v)
            pltpu.sync_copy(w1_hbm.at[pl.ds(t0, CHUNK)], w1_v)

            for slot in range(2):
                p_v = p0_v if slot == 0 else p1_v
                wsel_v = w0_v if slot == 0 else w1_v
                cpa = pltpu.async_copy(ys_hbm.at[p_v], tmpa_v, sema)
                for i in range(CHUNK // 16):
                    sl = pl.ds(i * 16, 16)
                    idx_v[sl] = p_v[sl] + G
                cpb = pltpu.async_copy(ys_hbm.at[idx_v], tmpb_v, semb)
                cpa.wait()
                cpb.wait()
                for g in range(CHUNK // 16):
                    wv = wsel_v[pl.ds(g * 16, 16)]
                    for li in range(16):
                        row = g * 16 + li
                        w = wv[li]

                        @pl.loop(0, D_MODEL // 16, unroll=4)
                        def _col(c):
                            sl = pl.ds(c * 16, 16)
                            contrib = w * (tmpa_v[row, sl] + tmpb_v[row, sl])
                            if slot == 0:
                                acc_v[row, sl] = contrib
                            else:
                                acc_v[row, sl] += contrib

            pltpu.sync_copy(acc_v, out_hbm.at[pl.ds(t0, CHUNK)])

    return _combine


def kernel(hidden_states, Wg, W1, W3, W2):
    b, s, d = hidden_states.shape
    x = hidden_states.reshape(TOKENS, D_MODEL)
    e1, e2, w1f, w2f = _router(x, Wg)
    e1 = e1[:, 0]
    e2 = e2[:, 0]
    pos0, pos1, block_expert = _metadata(e1, e2)
    xs = _dispatch_kernel()(x, pos0, pos1)
    ys = _ffn(block_expert, xs, W1, W3, W2)
    out = _combine_kernel()(ys, pos0, pos1, w1f[:, 0], w2f[:, 0])
    return out.reshape(b, s, d)


# speedup vs baseline: 1.2968x; 1.1148x over previous
import functools

import jax
import jax.numpy as jnp
from jax import lax
from jax.experimental import pallas as pl
from jax.experimental.pallas import tpu as pltpu
from jax.experimental.pallas import tpu_sc as plsc

D_MODEL = 1024
FFN = 3584
N_EXP = 8
TOKENS = 2048
PAIRS = 2 * TOKENS
BLK = 128
NBLK = 40
G = NBLK * BLK

NW = 32
TOK_W = TOKENS // NW
CHUNK = 32

@functools.cache
def _sc_mesh():
    return plsc.VectorSubcoreMesh(core_axis_name="c", subcore_axis_name="s")


def _router_body(x_ref, wg_ref, pos0_ref, pos1_ref, w1_ref, w2_ref, be_ref,
                 c1_ref, c2_ref):
    x = x_ref[...]
    wg = wg_ref[...]
    logits = lax.dot_general(x, wg, (((1,), (1,)), ((), ())),
                             preferred_element_type=jnp.float32)
    ids = lax.broadcasted_iota(jnp.int32, logits.shape, 1)
    m1 = jnp.max(logits, axis=1, keepdims=True)
    a1 = jnp.min(jnp.where(logits == m1, ids, N_EXP), axis=1, keepdims=True)
    masked = jnp.where(ids == a1, -jnp.inf, logits)
    m2 = jnp.max(masked, axis=1, keepdims=True)
    a2 = jnp.min(jnp.where(masked == m2, ids, N_EXP), axis=1, keepdims=True)
    w1_ref[...] = jax.nn.sigmoid(m1 - m2)
    w2_ref[...] = jax.nn.sigmoid(m2 - m1)

    oh1 = (ids == a1).astype(jnp.float32)
    oh2 = (ids == a2).astype(jnp.float32)
    rci = lax.broadcasted_iota(jnp.int32, (BLK, BLK), 0)
    cci = lax.broadcasted_iota(jnp.int32, (BLK, BLK), 1)
    texc = (rci > cci).astype(jnp.float32)
    base1 = jnp.zeros((1, N_EXP), jnp.float32)
    base2 = jnp.zeros((1, N_EXP), jnp.float32)
    for blk in range(TOKENS // BLK):
        sl = slice(blk * BLK, (blk + 1) * BLK)
        seg1 = oh1[sl, :]
        seg2 = oh2[sl, :]
        c1_ref[sl, :] = lax.dot_general(
            texc, seg1, (((1,), (0,)), ((), ())),
            preferred_element_type=jnp.float32) + base1
        c2_ref[sl, :] = lax.dot_general(
            texc, seg2, (((1,), (0,)), ((), ())),
            preferred_element_type=jnp.float32) + base2
        base1 = base1 + jnp.sum(seg1, axis=0, keepdims=True)
        base2 = base2 + jnp.sum(seg2, axis=0, keepdims=True)
    total1 = base1
    counts = base1 + base2
    nb = jnp.floor((counts + (BLK - 1)) * (1.0 / BLK))
    e8r = lax.broadcasted_iota(jnp.int32, (N_EXP, N_EXP), 0)
    e8c = lax.broadcasted_iota(jnp.int32, (N_EXP, N_EXP), 1)
    tincl8 = (e8r <= e8c).astype(jnp.float32)
    texc8 = (e8r < e8c).astype(jnp.float32)
    cumnb = lax.dot_general(nb, tincl8, (((1,), (0,)), ((), ())),
                            preferred_element_type=jnp.float32)
    astart = lax.dot_general(nb, texc8, (((1,), (0,)), ((), ())),
                             preferred_element_type=jnp.float32) * BLK
    pos0 = jnp.sum(oh1 * (astart + c1_ref[...]), axis=1, keepdims=True)
    pos1 = jnp.sum(oh2 * (astart + total1 + c2_ref[...]), axis=1, keepdims=True)
    pos0_ref[...] = pos0.astype(jnp.int32)
    pos1_ref[...] = pos1.astype(jnp.int32)
    bi = lax.broadcasted_iota(jnp.int32, (NBLK, N_EXP), 0).astype(jnp.float32)
    be = jnp.sum((jnp.broadcast_to(cumnb, (NBLK, N_EXP)) <= bi)
                 .astype(jnp.float32), axis=1, keepdims=True)
    be_ref[...] = jnp.minimum(be, N_EXP - 1).astype(jnp.int32)


def _router(x, Wg):
    return pl.pallas_call(
        _router_body,
        grid=(1,),
        in_specs=[pl.BlockSpec((TOKENS, D_MODEL), lambda b: (0, 0)),
                  pl.BlockSpec((N_EXP, D_MODEL), lambda b: (0, 0))],
        out_specs=[pl.BlockSpec((TOKENS, 1), lambda b: (0, 0)),
                   pl.BlockSpec((TOKENS, 1), lambda b: (0, 0)),
                   pl.BlockSpec((TOKENS, 1), lambda b: (0, 0)),
                   pl.BlockSpec((TOKENS, 1), lambda b: (0, 0)),
                   pl.BlockSpec((NBLK, 1), lambda b: (0, 0))],
        out_shape=[jax.ShapeDtypeStruct((TOKENS, 1), jnp.int32),
                   jax.ShapeDtypeStruct((TOKENS, 1), jnp.int32),
                   jax.ShapeDtypeStruct((TOKENS, 1), jnp.float32),
                   jax.ShapeDtypeStruct((TOKENS, 1), jnp.float32),
                   jax.ShapeDtypeStruct((NBLK, 1), jnp.int32)],
        scratch_shapes=[pltpu.VMEM((TOKENS, N_EXP), jnp.float32),
                        pltpu.VMEM((TOKENS, N_EXP), jnp.float32)],
    )(x, Wg)


@functools.cache
def _dispatch_kernel():
    @functools.partial(
        pl.kernel, mesh=_sc_mesh(),
        out_type=jax.ShapeDtypeStruct((G, D_MODEL), jnp.float32),
        scratch_types=[pltpu.VMEM((TOK_W,), jnp.int32),
                       pltpu.VMEM((TOK_W,), jnp.int32),
                       pltpu.VMEM((TOK_W, D_MODEL), jnp.float32),
                       pltpu.SemaphoreType.DMA,
                       pltpu.SemaphoreType.DMA])
    def _dispatch(x_hbm, pos0_hbm, pos1_hbm, xs_hbm, idx0_v, idx1_v, rows_v,
                  sem0, sem1):
        wid = lax.axis_index("s") * 2 + lax.axis_index("c")
        t0 = wid * TOK_W
        pltpu.sync_copy(x_hbm.at[pl.ds(t0, TOK_W)], rows_v)
        pltpu.sync_copy(pos0_hbm.at[pl.ds(t0, TOK_W)], idx0_v)
        pltpu.sync_copy(pos1_hbm.at[pl.ds(t0, TOK_W)], idx1_v)
        cp0 = pltpu.async_copy(rows_v, xs_hbm.at[idx0_v], sem0)
        cp1 = pltpu.async_copy(rows_v, xs_hbm.at[idx1_v], sem1)
        cp0.wait()
        cp1.wait()

    return _dispatch


NF = 2
FT = FFN // NF


def _ffn_body(be_ref, x_ref, w1_ref, w3_ref, w2_ref, o_ref):
    x = x_ref[...]
    a = lax.dot_general(x, w1_ref[0], (((1,), (1,)), ((), ())),
                        preferred_element_type=jnp.float32)
    b = lax.dot_general(x, w3_ref[0], (((1,), (1,)), ((), ())),
                        preferred_element_type=jnp.float32)
    h = a * jax.nn.sigmoid(a) * b
    o_ref[0] = lax.dot_general(h, w2_ref[0], (((1,), (1,)), ((), ())),
                               preferred_element_type=jnp.float32)


def _ffn(block_expert, xs, W1, W3, W2):
    grid_spec = pltpu.PrefetchScalarGridSpec(
        num_scalar_prefetch=1,
        grid=(NF, NBLK),
        in_specs=[
            pl.BlockSpec((BLK, D_MODEL), lambda f, b, be: (b, 0)),
            pl.BlockSpec((1, FT, D_MODEL), lambda f, b, be: (be[b], f, 0)),
            pl.BlockSpec((1, FT, D_MODEL), lambda f, b, be: (be[b], f, 0)),
            pl.BlockSpec((1, D_MODEL, FT), lambda f, b, be: (be[b], 0, f)),
        ],
        out_specs=pl.BlockSpec((1, BLK, D_MODEL), lambda f, b, be: (f, b, 0)),
    )
    ys = pl.pallas_call(
        _ffn_body,
        grid_spec=grid_spec,
        out_shape=jax.ShapeDtypeStruct((NF, G, D_MODEL), jnp.float32),
    )(block_expert, xs, W1, W3, W2)
    return ys.reshape(NF * G, D_MODEL)


@functools.cache
def _combine_kernel():
    @functools.partial(
        pl.kernel, mesh=_sc_mesh(),
        out_type=jax.ShapeDtypeStruct((TOKENS, D_MODEL), jnp.float32),
        scratch_types=[pltpu.VMEM((CHUNK,), jnp.int32),
                       pltpu.VMEM((CHUNK,), jnp.int32),
                       pltpu.VMEM((CHUNK,), jnp.int32),
                       pltpu.VMEM((CHUNK,), jnp.float32),
                       pltpu.VMEM((CHUNK,), jnp.float32),
                       pltpu.VMEM((CHUNK, D_MODEL), jnp.float32),
                       pltpu.VMEM((CHUNK, D_MODEL), jnp.float32),
                       pltpu.VMEM((CHUNK, D_MODEL), jnp.float32),
                       pltpu.SemaphoreType.DMA,
                       pltpu.SemaphoreType.DMA])
    def _combine(ys_hbm, pos0_hbm, pos1_hbm, w0_hbm, w1_hbm, out_hbm,
                 p0_v, p1_v, idx_v, w0_v, w1_v, acc_v, tmpa_v, tmpb_v,
                 sema, semb):
        wid = lax.axis_index("s") * 2 + lax.axis_index("c")
        for chunk in range(TOK_W // CHUNK):
            t0 = wid * TOK_W + chunk * CHUNK
            pltpu.sync_copy(pos0_hbm.at[pl.ds(t0, CHUNK)], p0_v)
            pltpu.sync_copy(pos1_hbm.at[pl.ds(t0, CHUNK)], p1_v)
            pltpu.sync_copy(w0_hbm.at[pl.ds(t0, CHUNK)], w0_v)
            pltpu.sync_copy(w1_hbm.at[pl.ds(t0, CHUNK)], w1_v)

            for slot in range(2):
                p_v = p0_v if slot == 0 else p1_v
                wsel_v = w0_v if slot == 0 else w1_v
                cpa = pltpu.async_copy(ys_hbm.at[p_v], tmpa_v, sema)
                for i in range(CHUNK // 16):
                    sl = pl.ds(i * 16, 16)
                    idx_v[sl] = p_v[sl] + G
                cpb = pltpu.async_copy(ys_hbm.at[idx_v], tmpb_v, semb)
                cpa.wait()
                cpb.wait()
                for g in range(CHUNK // 16):
                    wv = wsel_v[pl.ds(g * 16, 16)]
                    for li in range(16):
                        row = g * 16 + li
                        w = wv[li]

                        @pl.loop(0, D_MODEL // 16, unroll=4)
                        def _col(c):
                            sl = pl.ds(c * 16, 16)
                            contrib = w * (tmpa_v[row, sl] + tmpb_v[row, sl])
                            if slot == 0:
                                acc_v[row, sl] = contrib
                            else:
                                acc_v[row, sl] += contrib

            pltpu.sync_copy(acc_v, out_hbm.at[pl.ds(t0, CHUNK)])

    return _combine


def kernel(hidden_states, Wg, W1, W3, W2):
    b, s, d = hidden_states.shape
    x = hidden_states.reshape(TOKENS, D_MODEL)
    pos0, pos1, w1f, w2f, be = _router(x, Wg)
    pos0 = pos0[:, 0]
    pos1 = pos1[:, 0]
    block_expert = be[:, 0]
    xs = _dispatch_kernel()(x, pos0, pos1)
    ys = _ffn(block_expert, xs, W1, W3, W2)
    out = _combine_kernel()(ys, pos0, pos1, w1f[:, 0], w2f[:, 0])
    return out.reshape(b, s, d)


# speedup vs baseline: 1.3197x; 1.0176x over previous
import functools

import jax
import jax.numpy as jnp
from jax import lax
from jax.experimental import pallas as pl
from jax.experimental.pallas import tpu as pltpu
from jax.experimental.pallas import tpu_sc as plsc

D_MODEL = 1024
FFN = 3584
N_EXP = 8
TOKENS = 2048
PAIRS = 2 * TOKENS
BLK = 128
NBLK = 40
G = NBLK * BLK

NW = 32
TOK_W = TOKENS // NW
CHUNK = 16

@functools.cache
def _sc_mesh():
    return plsc.VectorSubcoreMesh(core_axis_name="c", subcore_axis_name="s")


def _router_body(x_ref, wg_ref, pos0_ref, pos1_ref, w1_ref, w2_ref, be_ref,
                 c1_ref, c2_ref):
    x = x_ref[...]
    wg = wg_ref[...]
    logits = lax.dot_general(x, wg, (((1,), (1,)), ((), ())),
                             preferred_element_type=jnp.float32)
    ids = lax.broadcasted_iota(jnp.int32, logits.shape, 1)
    m1 = jnp.max(logits, axis=1, keepdims=True)
    a1 = jnp.min(jnp.where(logits == m1, ids, N_EXP), axis=1, keepdims=True)
    masked = jnp.where(ids == a1, -jnp.inf, logits)
    m2 = jnp.max(masked, axis=1, keepdims=True)
    a2 = jnp.min(jnp.where(masked == m2, ids, N_EXP), axis=1, keepdims=True)
    w1_ref[...] = jax.nn.sigmoid(m1 - m2)
    w2_ref[...] = jax.nn.sigmoid(m2 - m1)

    oh1 = (ids == a1).astype(jnp.float32)
    oh2 = (ids == a2).astype(jnp.float32)
    rci = lax.broadcasted_iota(jnp.int32, (BLK, BLK), 0)
    cci = lax.broadcasted_iota(jnp.int32, (BLK, BLK), 1)
    texc = (rci > cci).astype(jnp.float32)
    base1 = jnp.zeros((1, N_EXP), jnp.float32)
    base2 = jnp.zeros((1, N_EXP), jnp.float32)
    for blk in range(TOKENS // BLK):
        sl = slice(blk * BLK, (blk + 1) * BLK)
        seg1 = oh1[sl, :]
        seg2 = oh2[sl, :]
        c1_ref[sl, :] = lax.dot_general(
            texc, seg1, (((1,), (0,)), ((), ())),
            preferred_element_type=jnp.float32) + base1
        c2_ref[sl, :] = lax.dot_general(
            texc, seg2, (((1,), (0,)), ((), ())),
            preferred_element_type=jnp.float32) + base2
        base1 = base1 + jnp.sum(seg1, axis=0, keepdims=True)
        base2 = base2 + jnp.sum(seg2, axis=0, keepdims=True)
    total1 = base1
    counts = base1 + base2
    nb = jnp.floor((counts + (BLK - 1)) * (1.0 / BLK))
    e8r = lax.broadcasted_iota(jnp.int32, (N_EXP, N_EXP), 0)
    e8c = lax.broadcasted_iota(jnp.int32, (N_EXP, N_EXP), 1)
    tincl8 = (e8r <= e8c).astype(jnp.float32)
    texc8 = (e8r < e8c).astype(jnp.float32)
    cumnb = lax.dot_general(nb, tincl8, (((1,), (0,)), ((), ())),
                            preferred_element_type=jnp.float32)
    astart = lax.dot_general(nb, texc8, (((1,), (0,)), ((), ())),
                             preferred_element_type=jnp.float32) * BLK
    pos0 = jnp.sum(oh1 * (astart + c1_ref[...]), axis=1, keepdims=True)
    pos1 = jnp.sum(oh2 * (astart + total1 + c2_ref[...]), axis=1, keepdims=True)
    pos0_ref[...] = pos0.astype(jnp.int32)
    pos1_ref[...] = pos1.astype(jnp.int32)
    bi = lax.broadcasted_iota(jnp.int32, (NBLK, N_EXP), 0).astype(jnp.float32)
    be = jnp.sum((jnp.broadcast_to(cumnb, (NBLK, N_EXP)) <= bi)
                 .astype(jnp.float32), axis=1, keepdims=True)
    be_ref[...] = jnp.minimum(be, N_EXP - 1).astype(jnp.int32)


def _router(x, Wg):
    return pl.pallas_call(
        _router_body,
        grid=(1,),
        in_specs=[pl.BlockSpec((TOKENS, D_MODEL), lambda b: (0, 0)),
                  pl.BlockSpec((N_EXP, D_MODEL), lambda b: (0, 0))],
        out_specs=[pl.BlockSpec((TOKENS, 1), lambda b: (0, 0)),
                   pl.BlockSpec((TOKENS, 1), lambda b: (0, 0)),
                   pl.BlockSpec((TOKENS, 1), lambda b: (0, 0)),
                   pl.BlockSpec((TOKENS, 1), lambda b: (0, 0)),
                   pl.BlockSpec((NBLK, 1), lambda b: (0, 0))],
        out_shape=[jax.ShapeDtypeStruct((TOKENS, 1), jnp.int32),
                   jax.ShapeDtypeStruct((TOKENS, 1), jnp.int32),
                   jax.ShapeDtypeStruct((TOKENS, 1), jnp.float32),
                   jax.ShapeDtypeStruct((TOKENS, 1), jnp.float32),
                   jax.ShapeDtypeStruct((NBLK, 1), jnp.int32)],
        scratch_shapes=[pltpu.VMEM((TOKENS, N_EXP), jnp.float32),
                        pltpu.VMEM((TOKENS, N_EXP), jnp.float32)],
    )(x, Wg)


@functools.cache
def _dispatch_kernel():
    @functools.partial(
        pl.kernel, mesh=_sc_mesh(),
        out_type=jax.ShapeDtypeStruct((G, D_MODEL), jnp.float32),
        scratch_types=[pltpu.VMEM((TOK_W,), jnp.int32),
                       pltpu.VMEM((TOK_W,), jnp.int32),
                       pltpu.VMEM((TOK_W, D_MODEL), jnp.float32),
                       pltpu.SemaphoreType.DMA,
                       pltpu.SemaphoreType.DMA])
    def _dispatch(x_hbm, pos0_hbm, pos1_hbm, xs_hbm, idx0_v, idx1_v, rows_v,
                  sem0, sem1):
        wid = lax.axis_index("s") * 2 + lax.axis_index("c")
        t0 = wid * TOK_W
        pltpu.sync_copy(x_hbm.at[pl.ds(t0, TOK_W)], rows_v)
        pltpu.sync_copy(pos0_hbm.at[pl.ds(t0, TOK_W)], idx0_v)
        pltpu.sync_copy(pos1_hbm.at[pl.ds(t0, TOK_W)], idx1_v)
        cp0 = pltpu.async_copy(rows_v, xs_hbm.at[idx0_v], sem0)
        cp1 = pltpu.async_copy(rows_v, xs_hbm.at[idx1_v], sem1)
        cp0.wait()
        cp1.wait()

    return _dispatch


NF = 2
FT = FFN // NF


def _ffn_body(be_ref, x_ref, w1_ref, w3_ref, w2_ref, o_ref):
    x = x_ref[...].astype(jnp.bfloat16)
    w1 = w1_ref[0].astype(jnp.bfloat16)
    w3 = w3_ref[0].astype(jnp.bfloat16)
    a = lax.dot_general(x, w1, (((1,), (1,)), ((), ())),
                        preferred_element_type=jnp.float32)
    b = lax.dot_general(x, w3, (((1,), (1,)), ((), ())),
                        preferred_element_type=jnp.float32)
    h = (a * jax.nn.sigmoid(a) * b).astype(jnp.bfloat16)
    w2 = w2_ref[0].astype(jnp.bfloat16)
    o_ref[0] = lax.dot_general(h, w2, (((1,), (1,)), ((), ())),
                               preferred_element_type=jnp.float32)


def _ffn(block_expert, xs, W1, W3, W2):
    grid_spec = pltpu.PrefetchScalarGridSpec(
        num_scalar_prefetch=1,
        grid=(NF, NBLK),
        in_specs=[
            pl.BlockSpec((BLK, D_MODEL), lambda f, b, be: (b, 0)),
            pl.BlockSpec((1, FT, D_MODEL), lambda f, b, be: (be[b], f, 0)),
            pl.BlockSpec((1, FT, D_MODEL), lambda f, b, be: (be[b], f, 0)),
            pl.BlockSpec((1, D_MODEL, FT), lambda f, b, be: (be[b], 0, f)),
        ],
        out_specs=pl.BlockSpec((1, BLK, D_MODEL), lambda f, b, be: (f, b, 0)),
    )
    ys = pl.pallas_call(
        _ffn_body,
        grid_spec=grid_spec,
        out_shape=jax.ShapeDtypeStruct((NF, G, D_MODEL), jnp.float32),
    )(block_expert, xs, W1, W3, W2)
    return ys.reshape(NF * G, D_MODEL)


@functools.cache
def _combine_kernel():
    @functools.partial(
        pl.kernel, mesh=_sc_mesh(),
        out_type=jax.ShapeDtypeStruct((TOKENS, D_MODEL), jnp.float32),
        scratch_types=[pltpu.VMEM((TOK_W,), jnp.int32),
                       pltpu.VMEM((TOK_W,), jnp.int32),
                       pltpu.VMEM((TOK_W,), jnp.float32),
                       pltpu.VMEM((TOK_W,), jnp.float32),
                       pltpu.VMEM((CHUNK, D_MODEL), jnp.float32),
                       pltpu.VMEM((CHUNK, D_MODEL), jnp.float32),
                       pltpu.VMEM((CHUNK, D_MODEL), jnp.float32),
                       pltpu.VMEM((CHUNK, D_MODEL), jnp.float32),
                       pltpu.SemaphoreType.DMA,
                       pltpu.SemaphoreType.DMA,
                       pltpu.SemaphoreType.DMA,
                       pltpu.SemaphoreType.DMA])
    def _combine(ys_hbm, pos0_hbm, pos1_hbm, w0_hbm, w1_hbm, out_hbm,
                 p0_v, p1_v, w0_v, w1_v, a0_v, b0_v, a1_v, b1_v,
                 sa0, sb0, sa1, sb1):
        wid = lax.axis_index("s") * 2 + lax.axis_index("c")
        t0 = wid * TOK_W
        pltpu.sync_copy(pos0_hbm.at[pl.ds(t0, TOK_W)], p0_v)
        pltpu.sync_copy(pos1_hbm.at[pl.ds(t0, TOK_W)], p1_v)
        pltpu.sync_copy(w0_hbm.at[pl.ds(t0, TOK_W)], w0_v)
        pltpu.sync_copy(w1_hbm.at[pl.ds(t0, TOK_W)], w1_v)
        for chunk in range(TOK_W // CHUNK):
            i0 = p0_v[pl.ds(chunk * CHUNK, CHUNK)]
            i1 = p1_v[pl.ds(chunk * CHUNK, CHUNK)]
            cpa0 = pltpu.async_copy(ys_hbm.at[i0], a0_v, sa0)
            cpb0 = pltpu.async_copy(ys_hbm.at[i0 + G], b0_v, sb0)
            cpa1 = pltpu.async_copy(ys_hbm.at[i1], a1_v, sa1)
            cpb1 = pltpu.async_copy(ys_hbm.at[i1 + G], b1_v, sb1)
            cpa0.wait()
            cpb0.wait()
            cpa1.wait()
            cpb1.wait()
            wv0 = w0_v[pl.ds(chunk * CHUNK, CHUNK)]
            wv1 = w1_v[pl.ds(chunk * CHUNK, CHUNK)]
            for li in range(CHUNK):
                w0 = wv0[li]
                w1 = wv1[li]

                @pl.loop(0, D_MODEL // 16, unroll=4)
                def _col(c):
                    sl = pl.ds(c * 16, 16)
                    a0_v[li, sl] = (w0 * (a0_v[li, sl] + b0_v[li, sl])
                                    + w1 * (a1_v[li, sl] + b1_v[li, sl]))

            pltpu.sync_copy(a0_v, out_hbm.at[pl.ds(t0 + chunk * CHUNK, CHUNK)])

    return _combine


def kernel(hidden_states, Wg, W1, W3, W2):
    b, s, d = hidden_states.shape
    x = hidden_states.reshape(TOKENS, D_MODEL)
    pos0, pos1, w1f, w2f, be = _router(x, Wg)
    pos0 = pos0[:, 0]
    pos1 = pos1[:, 0]
    block_expert = be[:, 0]
    xs = _dispatch_kernel()(x, pos0, pos1)
    ys = _ffn(block_expert, xs, W1, W3, W2)
    out = _combine_kernel()(ys, pos0, pos1, w1f[:, 0], w2f[:, 0])
    return out.reshape(b, s, d)
